# baseline (speedup 1.0000x reference)
import functools

import jax
import jax.numpy as jnp
from jax import lax
from jax.experimental import pallas as pl
from jax.experimental.pallas import tpu as pltpu
from jax.experimental.pallas import tpu_sc as plsc


def _cell_step(xp_ref, w_ref, u_ref, b_ref, li_ref, ri_ref,
               out_h_ref, out_c_ref, state_ref, g_ref, *, B, T, H):
    t = pl.program_id(0)

    @pl.when(t == 0)
    def _init():
        state_ref[T:T + 1, :, :] = jnp.zeros((1, B, 2 * H), jnp.float32)

    def gather_body(bi, carry):
        il = li_ref[t, bi]
        ir = ri_ref[t, bi]
        g_ref[0:1, pl.ds(bi, 1), :] = state_ref[pl.ds(il, 1), pl.ds(bi, 1), :]
        g_ref[1:2, pl.ds(bi, 1), :] = state_ref[pl.ds(ir, 1), pl.ds(bi, 1), :]
        return carry

    jax.lax.fori_loop(0, B, gather_body, 0, unroll=32)

    g = g_ref[...]
    hh = jnp.concatenate([g[0, :, :H], g[1, :, :H]], axis=-1)
    cl = g[0, :, H:]
    cr = g[1, :, H:]
    x = xp_ref[0]
    gates = (jnp.dot(x, w_ref[...], preferred_element_type=jnp.float32)
             + jnp.dot(hh, u_ref[...], preferred_element_type=jnp.float32)
             + b_ref[...])
    i_g = jax.nn.sigmoid(gates[:, 0:H])
    fl_g = jax.nn.sigmoid(gates[:, H:2 * H])
    fr_g = jax.nn.sigmoid(gates[:, 2 * H:3 * H])
    o_g = jax.nn.sigmoid(gates[:, 3 * H:4 * H])
    u_g = jnp.tanh(gates[:, 4 * H:5 * H])
    c_new = i_g * u_g + fl_g * cl + fr_g * cr
    h_new = o_g * jnp.tanh(c_new)
    hc = jnp.concatenate([h_new, c_new], axis=-1)
    state_ref[pl.ds(t, 1), :, :] = hc[None]
    out_h_ref[0:1, :, :] = h_new[None]
    out_c_ref[0:1, :, :] = c_new[None]


def _make_sc_expand(B, S, T, H):
    info = plsc.get_sparse_core_info()
    NC, NS = info.num_cores, info.num_subcores
    NW = NC * NS
    CH = 128
    ZCH = 256
    rows_per_w = (B * T) // NW
    n_chunks = rows_per_w // CH
    batches_per_w = B // NW
    z_per_b = S // ZCH
    mesh = plsc.VectorSubcoreMesh(core_axis_name="c", subcore_axis_name="s")

    @functools.partial(
        pl.kernel, mesh=mesh,
        out_type=[jax.ShapeDtypeStruct((B * S, H), jnp.float32),
                  jax.ShapeDtypeStruct((B * S, H), jnp.float32)],
        scratch_types=[
            pltpu.VMEM((ZCH, H), jnp.float32),
            pltpu.VMEM((CH,), jnp.int32),
            pltpu.VMEM((CH,), jnp.int32),
            pltpu.VMEM((CH, H), jnp.float32),
            pltpu.SemaphoreType.DMA,
        ],
    )
    def sc_expand(hrows_hbm, crows_hbm, src_hbm, tgt_hbm, zeros_hbm,
                  h_out, c_out, zbuf, sidx, tidx, rbuf, sem):
        wid = lax.axis_index("s") * NC + lax.axis_index("c")
        pltpu.sync_copy(zeros_hbm, zbuf)
        for k in range(batches_per_w):
            for j in range(z_per_b):
                row0 = (wid * batches_per_w + k) * S + j * ZCH
                pltpu.sync_copy(zbuf, h_out.at[pl.ds(row0, ZCH)])
                pltpu.sync_copy(zbuf, c_out.at[pl.ds(row0, ZCH)])
        for j in range(n_chunks):
            base = wid * rows_per_w + j * CH
            pltpu.sync_copy(src_hbm.at[pl.ds(base, CH)], sidx)
            pltpu.sync_copy(tgt_hbm.at[pl.ds(base, CH)], tidx)
            pltpu.async_copy(hrows_hbm.at[sidx], rbuf, sem).wait()
            pltpu.async_copy(rbuf, h_out.at[tidx], sem).wait()
            pltpu.async_copy(crows_hbm.at[sidx], rbuf, sem).wait()
            pltpu.async_copy(rbuf, c_out.at[tidx], sem).wait()

    return sc_expand


def kernel(input, tree_ids, W, U, b):
    B, S, E = input.shape
    T = tree_ids.shape[1]
    H = b.shape[0] // 5

    l = tree_ids[:, :, 0]
    r = tree_ids[:, :, 1]
    p = tree_ids[:, :, 2]

    tt = jnp.arange(T, dtype=jnp.int32)
    causal = (tt[None, :] < tt[:, None])[None]

    def last_writer(child, mask):
        eq = (p[:, None, :] == child[:, :, None]) & mask
        return jnp.max(jnp.where(eq, tt[None, None, :], -1), axis=-1)

    lwl = last_writer(l, causal)
    lwr = last_writer(r, causal)
    li = jnp.where(lwl < 0, T, lwl).astype(jnp.int32).T
    ri = jnp.where(lwr < 0, T, lwr).astype(jnp.int32).T

    fw = last_writer(p, True).astype(jnp.int32)
    brange = jnp.arange(B, dtype=jnp.int32)[:, None]
    src_flat = (fw * B + brange).reshape(-1)
    tgt_flat = (brange * S + p).reshape(-1)

    xp = jnp.take_along_axis(input, p[:, :, None], axis=1)
    xp = jnp.swapaxes(xp, 0, 1)
    b2 = b.reshape(1, 5 * H)

    h_comp, c_comp = pl.pallas_call(
        functools.partial(_cell_step, B=B, T=T, H=H),
        grid=(T,),
        in_specs=[
            pl.BlockSpec((1, B, E), lambda t: (t, 0, 0)),
            pl.BlockSpec((E, 5 * H), lambda t: (0, 0)),
            pl.BlockSpec((2 * H, 5 * H), lambda t: (0, 0)),
            pl.BlockSpec((1, 5 * H), lambda t: (0, 0)),
            pl.BlockSpec(memory_space=pltpu.SMEM),
            pl.BlockSpec(memory_space=pltpu.SMEM),
        ],
        out_specs=[
            pl.BlockSpec((1, B, H), lambda t: (t, 0, 0)),
            pl.BlockSpec((1, B, H), lambda t: (t, 0, 0)),
        ],
        out_shape=[
            jax.ShapeDtypeStruct((T, B, H), jnp.float32),
            jax.ShapeDtypeStruct((T, B, H), jnp.float32),
        ],
        scratch_shapes=[
            pltpu.VMEM((T + 1, B, 2 * H), jnp.float32),
            pltpu.VMEM((2, B, 2 * H), jnp.float32),
        ],
    )(xp, W, U, b2, li, ri)

    zeros_page = jnp.zeros((256, H), jnp.float32)
    h_flat, c_flat = _make_sc_expand(B, S, T, H)(
        h_comp.reshape(T * B, H), c_comp.reshape(T * B, H),
        src_flat, tgt_flat, zeros_page)
    return (h_flat.reshape(B, S, H), c_flat.reshape(B, S, H))


# speedup vs baseline: 1.1446x; 1.1446x over previous
import functools

import jax
import jax.numpy as jnp
from jax import lax
from jax.experimental import pallas as pl
from jax.experimental.pallas import tpu as pltpu
from jax.experimental.pallas import tpu_sc as plsc


def _cell_step(xp_ref, w_ref, u_ref, b_ref, li_ref, ri_ref,
               out_h_ref, out_c_ref, state_ref, g_ref, *, B, T, H):
    t = pl.program_id(0)

    @pl.when(t == 0)
    def _init():
        state_ref[T:T + 1, :, :] = jnp.zeros((1, B, 2 * H), jnp.float32)

    for bi in range(B):
        il = li_ref[t, bi]
        ir = ri_ref[t, bi]
        g_ref[0:1, bi:bi + 1, :] = state_ref[pl.ds(il, 1), bi:bi + 1, :]
        g_ref[1:2, bi:bi + 1, :] = state_ref[pl.ds(ir, 1), bi:bi + 1, :]

    g = g_ref[...]
    hh = jnp.concatenate([g[0, :, :H], g[1, :, :H]], axis=-1)
    cl = g[0, :, H:]
    cr = g[1, :, H:]
    x = xp_ref[0]
    gates = (jnp.dot(x, w_ref[...], preferred_element_type=jnp.float32)
             + jnp.dot(hh, u_ref[...], preferred_element_type=jnp.float32)
             + b_ref[...])
    i_g = jax.nn.sigmoid(gates[:, 0:H])
    fl_g = jax.nn.sigmoid(gates[:, H:2 * H])
    fr_g = jax.nn.sigmoid(gates[:, 2 * H:3 * H])
    o_g = jax.nn.sigmoid(gates[:, 3 * H:4 * H])
    u_g = jnp.tanh(gates[:, 4 * H:5 * H])
    c_new = i_g * u_g + fl_g * cl + fr_g * cr
    h_new = o_g * jnp.tanh(c_new)
    hc = jnp.concatenate([h_new, c_new], axis=-1)
    state_ref[pl.ds(t, 1), :, :] = hc[None]
    out_h_ref[0:1, :, :] = h_new[None]
    out_c_ref[0:1, :, :] = c_new[None]


def _make_sc_expand(B, S, T, H):
    info = plsc.get_sparse_core_info()
    NC, NS = info.num_cores, info.num_subcores
    NW = NC * NS
    CH = 128
    ZCH = 256
    rows_per_w = (B * T) // NW
    n_chunks = rows_per_w // CH
    batches_per_w = B // NW
    z_per_b = S // ZCH
    mesh = plsc.VectorSubcoreMesh(core_axis_name="c", subcore_axis_name="s")

    @functools.partial(
        pl.kernel, mesh=mesh,
        out_type=[jax.ShapeDtypeStruct((B * S, H), jnp.float32),
                  jax.ShapeDtypeStruct((B * S, H), jnp.float32)],
        scratch_types=[
            pltpu.VMEM((ZCH, H), jnp.float32),
            pltpu.VMEM((CH,), jnp.int32),
            pltpu.VMEM((CH,), jnp.int32),
            pltpu.VMEM((CH, H), jnp.float32),
            pltpu.SemaphoreType.DMA,
        ],
    )
    def sc_expand(hrows_hbm, crows_hbm, src_hbm, tgt_hbm, zeros_hbm,
                  h_out, c_out, zbuf, sidx, tidx, rbuf, sem):
        wid = lax.axis_index("s") * NC + lax.axis_index("c")
        pltpu.sync_copy(zeros_hbm, zbuf)
        for k in range(batches_per_w):
            for j in range(z_per_b):
                row0 = (wid * batches_per_w + k) * S + j * ZCH
                pltpu.sync_copy(zbuf, h_out.at[pl.ds(row0, ZCH)])
                pltpu.sync_copy(zbuf, c_out.at[pl.ds(row0, ZCH)])
        for j in range(n_chunks):
            base = wid * rows_per_w + j * CH
            pltpu.sync_copy(src_hbm.at[pl.ds(base, CH)], sidx)
            pltpu.sync_copy(tgt_hbm.at[pl.ds(base, CH)], tidx)
            pltpu.async_copy(hrows_hbm.at[sidx], rbuf, sem).wait()
            pltpu.async_copy(rbuf, h_out.at[tidx], sem).wait()
            pltpu.async_copy(crows_hbm.at[sidx], rbuf, sem).wait()
            pltpu.async_copy(rbuf, c_out.at[tidx], sem).wait()

    return sc_expand


def kernel(input, tree_ids, W, U, b):
    B, S, E = input.shape
    T = tree_ids.shape[1]
    H = b.shape[0] // 5

    l = tree_ids[:, :, 0]
    r = tree_ids[:, :, 1]
    p = tree_ids[:, :, 2]

    tt = jnp.arange(T, dtype=jnp.int32)
    causal = (tt[None, :] < tt[:, None])[None]

    def last_writer(child, mask):
        eq = (p[:, None, :] == child[:, :, None]) & mask
        return jnp.max(jnp.where(eq, tt[None, None, :], -1), axis=-1)

    lwl = last_writer(l, causal)
    lwr = last_writer(r, causal)
    li = jnp.where(lwl < 0, T, lwl).astype(jnp.int32).T
    ri = jnp.where(lwr < 0, T, lwr).astype(jnp.int32).T

    fw = last_writer(p, True).astype(jnp.int32)
    brange = jnp.arange(B, dtype=jnp.int32)[:, None]
    src_flat = (fw * B + brange).reshape(-1)
    tgt_flat = (brange * S + p).reshape(-1)

    xp = jnp.take_along_axis(input, p[:, :, None], axis=1)
    xp = jnp.swapaxes(xp, 0, 1)
    b2 = b.reshape(1, 5 * H)

    h_comp, c_comp = pl.pallas_call(
        functools.partial(_cell_step, B=B, T=T, H=H),
        grid=(T,),
        in_specs=[
            pl.BlockSpec((1, B, E), lambda t: (t, 0, 0)),
            pl.BlockSpec((E, 5 * H), lambda t: (0, 0)),
            pl.BlockSpec((2 * H, 5 * H), lambda t: (0, 0)),
            pl.BlockSpec((1, 5 * H), lambda t: (0, 0)),
            pl.BlockSpec(memory_space=pltpu.SMEM),
            pl.BlockSpec(memory_space=pltpu.SMEM),
        ],
        out_specs=[
            pl.BlockSpec((1, B, H), lambda t: (t, 0, 0)),
            pl.BlockSpec((1, B, H), lambda t: (t, 0, 0)),
        ],
        out_shape=[
            jax.ShapeDtypeStruct((T, B, H), jnp.float32),
            jax.ShapeDtypeStruct((T, B, H), jnp.float32),
        ],
        scratch_shapes=[
            pltpu.VMEM((T + 1, B, 2 * H), jnp.float32),
            pltpu.VMEM((2, B, 2 * H), jnp.float32),
        ],
    )(xp, W, U, b2, li, ri)

    zeros_page = jnp.zeros((256, H), jnp.float32)
    h_flat, c_flat = _make_sc_expand(B, S, T, H)(
        h_comp.reshape(T * B, H), c_comp.reshape(T * B, H),
        src_flat, tgt_flat, zeros_page)
    return (h_flat.reshape(B, S, H), c_flat.reshape(B, S, H))


# speedup vs baseline: 1.1857x; 1.0359x over previous
import functools

import jax
import jax.numpy as jnp
from jax import lax
from jax.experimental import pallas as pl
from jax.experimental.pallas import tpu as pltpu
from jax.experimental.pallas import tpu_sc as plsc


def _cell_step(xp_ref, w_ref, u_ref, b_ref, li_ref, ri_ref,
               out_h_ref, out_c_ref, state_ref, g_ref, *, B, T, H):
    t = pl.program_id(0)

    @pl.when(t == 0)
    def _init():
        state_ref[T:T + 1, :, :] = jnp.zeros((1, B, 2 * H), jnp.float32)

    x = xp_ref[0]
    xw = jnp.dot(x, w_ref[...], preferred_element_type=jnp.float32) + b_ref[...]

    for bi in range(B):
        il = li_ref[t, bi]
        g_ref[0:1, bi:bi + 1, :] = state_ref[pl.ds(il, 1), bi:bi + 1, :]
    g0 = g_ref[0]
    hl = g0[:, :H]
    cl = g0[:, H:]
    gl = jnp.dot(hl, u_ref[0:H, :], preferred_element_type=jnp.float32)
    for bi in range(B):
        ir = ri_ref[t, bi]
        g_ref[1:2, bi:bi + 1, :] = state_ref[pl.ds(ir, 1), bi:bi + 1, :]
    g1 = g_ref[1]
    hr = g1[:, :H]
    cr = g1[:, H:]
    gates = (xw + gl
             + jnp.dot(hr, u_ref[H:2 * H, :], preferred_element_type=jnp.float32))
    i_g = jax.nn.sigmoid(gates[:, 0:H])
    fl_g = jax.nn.sigmoid(gates[:, H:2 * H])
    fr_g = jax.nn.sigmoid(gates[:, 2 * H:3 * H])
    o_g = jax.nn.sigmoid(gates[:, 3 * H:4 * H])
    u_g = jnp.tanh(gates[:, 4 * H:5 * H])
    c_new = i_g * u_g + fl_g * cl + fr_g * cr
    h_new = o_g * jnp.tanh(c_new)
    hc = jnp.concatenate([h_new, c_new], axis=-1)
    state_ref[pl.ds(t, 1), :, :] = hc[None]
    out_h_ref[0:1, :, :] = h_new[None]
    out_c_ref[0:1, :, :] = c_new[None]


def _make_sc_expand(B, S, T, H):
    info = plsc.get_sparse_core_info()
    NC, NS = info.num_cores, info.num_subcores
    NW = NC * NS
    CH = 128
    ZCH = 256
    rows_per_w = (B * T) // NW
    n_chunks = rows_per_w // CH
    batches_per_w = B // NW
    z_per_b = S // ZCH
    mesh = plsc.VectorSubcoreMesh(core_axis_name="c", subcore_axis_name="s")

    @functools.partial(
        pl.kernel, mesh=mesh,
        out_type=[jax.ShapeDtypeStruct((B * S, H), jnp.float32),
                  jax.ShapeDtypeStruct((B * S, H), jnp.float32)],
        scratch_types=[
            pltpu.VMEM((ZCH, H), jnp.float32),
            pltpu.VMEM((CH,), jnp.int32),
            pltpu.VMEM((CH,), jnp.int32),
            pltpu.VMEM((CH, H), jnp.float32),
            pltpu.SemaphoreType.DMA,
        ],
    )
    def sc_expand(hrows_hbm, crows_hbm, src_hbm, tgt_hbm, zeros_hbm,
                  h_out, c_out, zbuf, sidx, tidx, rbuf, sem):
        wid = lax.axis_index("s") * NC + lax.axis_index("c")
        pltpu.sync_copy(zeros_hbm, zbuf)
        for k in range(batches_per_w):
            for j in range(z_per_b):
                row0 = (wid * batches_per_w + k) * S + j * ZCH
                pltpu.sync_copy(zbuf, h_out.at[pl.ds(row0, ZCH)])
                pltpu.sync_copy(zbuf, c_out.at[pl.ds(row0, ZCH)])
        for j in range(n_chunks):
            base = wid * rows_per_w + j * CH
            pltpu.sync_copy(src_hbm.at[pl.ds(base, CH)], sidx)
            pltpu.sync_copy(tgt_hbm.at[pl.ds(base, CH)], tidx)
            pltpu.async_copy(hrows_hbm.at[sidx], rbuf, sem).wait()
            pltpu.async_copy(rbuf, h_out.at[tidx], sem).wait()
            pltpu.async_copy(crows_hbm.at[sidx], rbuf, sem).wait()
            pltpu.async_copy(rbuf, c_out.at[tidx], sem).wait()

    return sc_expand


def kernel(input, tree_ids, W, U, b):
    B, S, E = input.shape
    T = tree_ids.shape[1]
    H = b.shape[0] // 5

    l = tree_ids[:, :, 0]
    r = tree_ids[:, :, 1]
    p = tree_ids[:, :, 2]

    tt = jnp.arange(T, dtype=jnp.int32)
    causal = (tt[None, :] < tt[:, None])[None]

    def last_writer(child, mask):
        eq = (p[:, None, :] == child[:, :, None]) & mask
        return jnp.max(jnp.where(eq, tt[None, None, :], -1), axis=-1)

    lwl = last_writer(l, causal)
    lwr = last_writer(r, causal)
    li = jnp.where(lwl < 0, T, lwl).astype(jnp.int32).T
    ri = jnp.where(lwr < 0, T, lwr).astype(jnp.int32).T

    fw = last_writer(p, True).astype(jnp.int32)
    brange = jnp.arange(B, dtype=jnp.int32)[:, None]
    src_flat = (fw * B + brange).reshape(-1)
    tgt_flat = (brange * S + p).reshape(-1)

    xp = jnp.take_along_axis(input, p[:, :, None], axis=1)
    xp = jnp.swapaxes(xp, 0, 1)
    b2 = b.reshape(1, 5 * H)

    h_comp, c_comp = pl.pallas_call(
        functools.partial(_cell_step, B=B, T=T, H=H),
        grid=(T,),
        in_specs=[
            pl.BlockSpec((1, B, E), lambda t: (t, 0, 0)),
            pl.BlockSpec((E, 5 * H), lambda t: (0, 0)),
            pl.BlockSpec((2 * H, 5 * H), lambda t: (0, 0)),
            pl.BlockSpec((1, 5 * H), lambda t: (0, 0)),
            pl.BlockSpec(memory_space=pltpu.SMEM),
            pl.BlockSpec(memory_space=pltpu.SMEM),
        ],
        out_specs=[
            pl.BlockSpec((1, B, H), lambda t: (t, 0, 0)),
            pl.BlockSpec((1, B, H), lambda t: (t, 0, 0)),
        ],
        out_shape=[
            jax.ShapeDtypeStruct((T, B, H), jnp.float32),
            jax.ShapeDtypeStruct((T, B, H), jnp.float32),
        ],
        scratch_shapes=[
            pltpu.VMEM((T + 1, B, 2 * H), jnp.float32),
            pltpu.VMEM((2, B, 2 * H), jnp.float32),
        ],
    )(xp, W, U, b2, li, ri)

    zeros_page = jnp.zeros((256, H), jnp.float32)
    h_flat, c_flat = _make_sc_expand(B, S, T, H)(
        h_comp.reshape(T * B, H), c_comp.reshape(T * B, H),
        src_flat, tgt_flat, zeros_page)
    return (h_flat.reshape(B, S, H), c_flat.reshape(B, S, H))


# speedup vs baseline: 1.2216x; 1.0302x over previous
import functools

import jax
import jax.numpy as jnp
from jax import lax
from jax.experimental import pallas as pl
from jax.experimental.pallas import tpu as pltpu
from jax.experimental.pallas import tpu_sc as plsc


def _cell_step(xp_ref, w_ref, u_ref, b_ref, li_ref, ri_ref,
               out_h_ref, out_c_ref, state_ref, g_ref, *, B, T, H):
    t = pl.program_id(0)

    @pl.when(t == 0)
    def _init():
        state_ref[T:T + 1, :, :] = jnp.zeros((1, B, 2 * H), jnp.float32)

    x = xp_ref[0]
    xw = jnp.dot(x, w_ref[...], preferred_element_type=jnp.float32) + b_ref[...]

    for bi in range(B):
        il = li_ref[t, bi]
        g_ref[0:1, bi:bi + 1, :] = state_ref[pl.ds(il, 1), bi:bi + 1, :]
    g0 = g_ref[0]
    hl = g0[:, :H]
    cl = g0[:, H:]
    gl = jnp.dot(hl, u_ref[0:H, :], preferred_element_type=jnp.float32)
    for bi in range(B):
        ir = ri_ref[t, bi]
        g_ref[1:2, bi:bi + 1, :] = state_ref[pl.ds(ir, 1), bi:bi + 1, :]
    g1 = g_ref[1]
    hr = g1[:, :H]
    cr = g1[:, H:]
    gates = (xw + gl
             + jnp.dot(hr, u_ref[H:2 * H, :], preferred_element_type=jnp.float32))
    i_g = jax.nn.sigmoid(gates[:, 0:H])
    fl_g = jax.nn.sigmoid(gates[:, H:2 * H])
    fr_g = jax.nn.sigmoid(gates[:, 2 * H:3 * H])
    o_g = jax.nn.sigmoid(gates[:, 3 * H:4 * H])
    u_g = jnp.tanh(gates[:, 4 * H:5 * H])
    c_new = i_g * u_g + fl_g * cl + fr_g * cr
    h_new = o_g * jnp.tanh(c_new)
    hc = jnp.concatenate([h_new, c_new], axis=-1)
    state_ref[pl.ds(t, 1), :, :] = hc[None]
    out_h_ref[0:1, :, :] = h_new[None]
    out_c_ref[0:1, :, :] = c_new[None]


def _make_sc_gather(B, S, T, E):
    info = plsc.get_sparse_core_info()
    NW = info.num_cores * info.num_subcores
    CH = 128
    rows_per_w = (B * T) // NW
    n_chunks = rows_per_w // CH
    mesh = plsc.VectorSubcoreMesh(core_axis_name="c", subcore_axis_name="s")

    @functools.partial(
        pl.kernel, mesh=mesh,
        out_type=jax.ShapeDtypeStruct((T * B, E), jnp.float32),
        scratch_types=[
            pltpu.VMEM((CH,), jnp.int32),
            pltpu.VMEM((CH, E), jnp.float32),
            pltpu.SemaphoreType.DMA,
        ],
    )
    def sc_gather(table_hbm, idx_hbm, out_hbm, iidx, buf, sem):
        wid = lax.axis_index("s") * info.num_cores + lax.axis_index("c")
        for j in range(n_chunks):
            base = wid * rows_per_w + j * CH
            pltpu.sync_copy(idx_hbm.at[pl.ds(base, CH)], iidx)
            pltpu.async_copy(table_hbm.at[iidx], buf, sem).wait()
            pltpu.sync_copy(buf, out_hbm.at[pl.ds(base, CH)])

    return sc_gather


def _make_sc_expand(B, S, T, H):
    info = plsc.get_sparse_core_info()
    NC, NS = info.num_cores, info.num_subcores
    NW = NC * NS
    CH = 128
    ZCH = 256
    rows_per_w = (B * T) // NW
    n_chunks = rows_per_w // CH
    batches_per_w = B // NW
    z_per_b = S // ZCH
    mesh = plsc.VectorSubcoreMesh(core_axis_name="c", subcore_axis_name="s")

    @functools.partial(
        pl.kernel, mesh=mesh,
        out_type=[jax.ShapeDtypeStruct((B * S, H), jnp.float32),
                  jax.ShapeDtypeStruct((B * S, H), jnp.float32)],
        scratch_types=[
            pltpu.VMEM((ZCH, H), jnp.float32),
            pltpu.VMEM((CH,), jnp.int32),
            pltpu.VMEM((CH,), jnp.int32),
            pltpu.VMEM((CH, H), jnp.float32),
            pltpu.SemaphoreType.DMA,
        ],
    )
    def sc_expand(hrows_hbm, crows_hbm, src_hbm, tgt_hbm, zeros_hbm,
                  h_out, c_out, zbuf, sidx, tidx, rbuf, sem):
        wid = lax.axis_index("s") * NC + lax.axis_index("c")
        pltpu.sync_copy(zeros_hbm, zbuf)
        for k in range(batches_per_w):
            for j in range(z_per_b):
                row0 = (wid * batches_per_w + k) * S + j * ZCH
                pltpu.sync_copy(zbuf, h_out.at[pl.ds(row0, ZCH)])
                pltpu.sync_copy(zbuf, c_out.at[pl.ds(row0, ZCH)])
        for j in range(n_chunks):
            base = wid * rows_per_w + j * CH
            pltpu.sync_copy(src_hbm.at[pl.ds(base, CH)], sidx)
            pltpu.sync_copy(tgt_hbm.at[pl.ds(base, CH)], tidx)
            pltpu.async_copy(hrows_hbm.at[sidx], rbuf, sem).wait()
            pltpu.async_copy(rbuf, h_out.at[tidx], sem).wait()
            pltpu.async_copy(crows_hbm.at[sidx], rbuf, sem).wait()
            pltpu.async_copy(rbuf, c_out.at[tidx], sem).wait()

    return sc_expand


def kernel(input, tree_ids, W, U, b):
    B, S, E = input.shape
    T = tree_ids.shape[1]
    H = b.shape[0] // 5

    l = tree_ids[:, :, 0]
    r = tree_ids[:, :, 1]
    p = tree_ids[:, :, 2]

    tt = jnp.arange(T, dtype=jnp.int32)
    causal = (tt[None, :] < tt[:, None])[None]

    def last_writer(child, mask):
        eq = (p[:, None, :] == child[:, :, None]) & mask
        return jnp.max(jnp.where(eq, tt[None, None, :], -1), axis=-1)

    lwl = last_writer(l, causal)
    lwr = last_writer(r, causal)
    li = jnp.where(lwl < 0, T, lwl).astype(jnp.int32).T
    ri = jnp.where(lwr < 0, T, lwr).astype(jnp.int32).T

    fw = last_writer(p, True).astype(jnp.int32)
    brange = jnp.arange(B, dtype=jnp.int32)[:, None]
    src_flat = (fw * B + brange).reshape(-1)
    tgt_flat = (brange * S + p).reshape(-1)

    xp_idx = (jnp.arange(B, dtype=jnp.int32)[None, :] * S + p.T).reshape(-1)
    xp = _make_sc_gather(B, S, T, E)(
        input.reshape(B * S, E), xp_idx).reshape(T, B, E)
    b2 = b.reshape(1, 5 * H)

    h_comp, c_comp = pl.pallas_call(
        functools.partial(_cell_step, B=B, T=T, H=H),
        grid=(T,),
        in_specs=[
            pl.BlockSpec((1, B, E), lambda t: (t, 0, 0)),
            pl.BlockSpec((E, 5 * H), lambda t: (0, 0)),
            pl.BlockSpec((2 * H, 5 * H), lambda t: (0, 0)),
            pl.BlockSpec((1, 5 * H), lambda t: (0, 0)),
            pl.BlockSpec(memory_space=pltpu.SMEM),
            pl.BlockSpec(memory_space=pltpu.SMEM),
        ],
        out_specs=[
            pl.BlockSpec((1, B, H), lambda t: (t, 0, 0)),
            pl.BlockSpec((1, B, H), lambda t: (t, 0, 0)),
        ],
        out_shape=[
            jax.ShapeDtypeStruct((T, B, H), jnp.float32),
            jax.ShapeDtypeStruct((T, B, H), jnp.float32),
        ],
        scratch_shapes=[
            pltpu.VMEM((T + 1, B, 2 * H), jnp.float32),
            pltpu.VMEM((2, B, 2 * H), jnp.float32),
        ],
    )(xp, W, U, b2, li, ri)

    zeros_page = jnp.zeros((256, H), jnp.float32)
    h_flat, c_flat = _make_sc_expand(B, S, T, H)(
        h_comp.reshape(T * B, H), c_comp.reshape(T * B, H),
        src_flat, tgt_flat, zeros_page)
    return (h_flat.reshape(B, S, H), c_flat.reshape(B, S, H))


# speedup vs baseline: 1.2439x; 1.0183x over previous
import functools

import jax
import jax.numpy as jnp
from jax import lax
from jax.experimental import pallas as pl
from jax.experimental.pallas import tpu as pltpu
from jax.experimental.pallas import tpu_sc as plsc


def _cell_step(xp_ref, w_ref, u_ref, b_ref, li_ref, ri_ref,
               out_h_ref, out_c_ref, state_ref, g_ref, *, B, T, H):
    t = pl.program_id(0)

    @pl.when(t == 0)
    def _init():
        state_ref[T:T + 1, :, :] = jnp.zeros((1, B, 2 * H), jnp.float32)

    x = xp_ref[0]
    xw = jnp.dot(x, w_ref[...], preferred_element_type=jnp.float32) + b_ref[...]

    for bi in range(B):
        il = li_ref[t, bi]
        g_ref[0:1, bi:bi + 1, :] = state_ref[pl.ds(il, 1), bi:bi + 1, :]
    g0 = g_ref[0]
    hl = g0[:, :H]
    cl = g0[:, H:]
    gl = jnp.dot(hl, u_ref[0:H, :], preferred_element_type=jnp.float32)
    for bi in range(B):
        ir = ri_ref[t, bi]
        g_ref[1:2, bi:bi + 1, :] = state_ref[pl.ds(ir, 1), bi:bi + 1, :]
    g1 = g_ref[1]
    hr = g1[:, :H]
    cr = g1[:, H:]
    gates = (xw + gl
             + jnp.dot(hr, u_ref[H:2 * H, :], preferred_element_type=jnp.float32))
    i_g = jax.nn.sigmoid(gates[:, 0:H])
    fl_g = jax.nn.sigmoid(gates[:, H:2 * H])
    fr_g = jax.nn.sigmoid(gates[:, 2 * H:3 * H])
    o_g = jax.nn.sigmoid(gates[:, 3 * H:4 * H])
    u_g = jnp.tanh(gates[:, 4 * H:5 * H])
    c_new = i_g * u_g + fl_g * cl + fr_g * cr
    h_new = o_g * jnp.tanh(c_new)
    hc = jnp.concatenate([h_new, c_new], axis=-1)
    state_ref[pl.ds(t, 1), :, :] = hc[None]
    out_h_ref[0:1, :, :] = h_new[None]
    out_c_ref[0:1, :, :] = c_new[None]


def _make_sc_gather(B, S, T, E):
    info = plsc.get_sparse_core_info()
    NW = info.num_cores * info.num_subcores
    CH = 128
    rows_per_w = (B * T) // NW
    n_chunks = rows_per_w // CH
    mesh = plsc.VectorSubcoreMesh(core_axis_name="c", subcore_axis_name="s")

    @functools.partial(
        pl.kernel, mesh=mesh,
        out_type=jax.ShapeDtypeStruct((T * B, E), jnp.float32),
        scratch_types=[
            pltpu.VMEM((CH,), jnp.int32),
            pltpu.VMEM((CH, E), jnp.float32),
            pltpu.SemaphoreType.DMA,
        ],
    )
    def sc_gather(table_hbm, idx_hbm, out_hbm, iidx, buf, sem):
        wid = lax.axis_index("s") * info.num_cores + lax.axis_index("c")
        for j in range(n_chunks):
            base = wid * rows_per_w + j * CH
            pltpu.sync_copy(idx_hbm.at[pl.ds(base, CH)], iidx)
            pltpu.async_copy(table_hbm.at[iidx], buf, sem).wait()
            pltpu.sync_copy(buf, out_hbm.at[pl.ds(base, CH)])

    return sc_gather


def _make_sc_expand(B, S, T, H):
    info = plsc.get_sparse_core_info()
    NC, NS = info.num_cores, info.num_subcores
    NW = NC * NS
    CH = 128
    ZCH = 256
    rows_per_w = (B * T) // NW
    n_chunks = rows_per_w // CH
    batches_per_w = B // NW
    z_per_b = S // ZCH
    mesh = plsc.VectorSubcoreMesh(core_axis_name="c", subcore_axis_name="s")

    @functools.partial(
        pl.kernel, mesh=mesh,
        out_type=[jax.ShapeDtypeStruct((B * S, H), jnp.float32),
                  jax.ShapeDtypeStruct((B * S, H), jnp.float32)],
        scratch_types=[
            pltpu.VMEM((ZCH, H), jnp.float32),
            pltpu.VMEM((CH,), jnp.int32),
            pltpu.VMEM((CH,), jnp.int32),
            pltpu.VMEM((CH, H), jnp.float32),
            pltpu.VMEM((CH, H), jnp.float32),
            pltpu.SemaphoreType.DMA,
            pltpu.SemaphoreType.DMA,
            pltpu.SemaphoreType.DMA,
        ],
    )
    def sc_expand(hrows_hbm, crows_hbm, src_hbm, tgt_hbm, zeros_hbm,
                  h_out, c_out, zbuf, sidx, tidx, rbufh, rbufc,
                  semz, semh, semc):
        wid = lax.axis_index("s") * NC + lax.axis_index("c")
        pltpu.sync_copy(zeros_hbm, zbuf)
        zcopies = []
        for k in range(batches_per_w):
            for j in range(z_per_b):
                row0 = (wid * batches_per_w + k) * S + j * ZCH
                zcopies.append(
                    pltpu.async_copy(zbuf, h_out.at[pl.ds(row0, ZCH)], semz))
                zcopies.append(
                    pltpu.async_copy(zbuf, c_out.at[pl.ds(row0, ZCH)], semz))
        for cp in zcopies:
            cp.wait()
        for j in range(n_chunks):
            base = wid * rows_per_w + j * CH
            pltpu.sync_copy(src_hbm.at[pl.ds(base, CH)], sidx)
            pltpu.sync_copy(tgt_hbm.at[pl.ds(base, CH)], tidx)
            gh = pltpu.async_copy(hrows_hbm.at[sidx], rbufh, semh)
            gc = pltpu.async_copy(crows_hbm.at[sidx], rbufc, semc)
            gh.wait()
            sh = pltpu.async_copy(rbufh, h_out.at[tidx], semh)
            gc.wait()
            sc2 = pltpu.async_copy(rbufc, c_out.at[tidx], semc)
            sh.wait()
            sc2.wait()

    return sc_expand


def kernel(input, tree_ids, W, U, b):
    B, S, E = input.shape
    T = tree_ids.shape[1]
    H = b.shape[0] // 5

    l = tree_ids[:, :, 0]
    r = tree_ids[:, :, 1]
    p = tree_ids[:, :, 2]

    tt = jnp.arange(T, dtype=jnp.int32)
    causal = (tt[None, :] < tt[:, None])[None]

    def last_writer(child, mask):
        eq = (p[:, None, :] == child[:, :, None]) & mask
        return jnp.max(jnp.where(eq, tt[None, None, :], -1), axis=-1)

    lwl = last_writer(l, causal)
    lwr = last_writer(r, causal)
    li = jnp.where(lwl < 0, T, lwl).astype(jnp.int32).T
    ri = jnp.where(lwr < 0, T, lwr).astype(jnp.int32).T

    fw = last_writer(p, True).astype(jnp.int32)
    brange = jnp.arange(B, dtype=jnp.int32)[:, None]
    src_flat = (fw * B + brange).reshape(-1)
    tgt_flat = (brange * S + p).reshape(-1)

    xp_idx = (jnp.arange(B, dtype=jnp.int32)[None, :] * S + p.T).reshape(-1)
    xp = _make_sc_gather(B, S, T, E)(
        input.reshape(B * S, E), xp_idx).reshape(T, B, E)
    b2 = b.reshape(1, 5 * H)

    h_comp, c_comp = pl.pallas_call(
        functools.partial(_cell_step, B=B, T=T, H=H),
        grid=(T,),
        in_specs=[
            pl.BlockSpec((1, B, E), lambda t: (t, 0, 0)),
            pl.BlockSpec((E, 5 * H), lambda t: (0, 0)),
            pl.BlockSpec((2 * H, 5 * H), lambda t: (0, 0)),
            pl.BlockSpec((1, 5 * H), lambda t: (0, 0)),
            pl.BlockSpec(memory_space=pltpu.SMEM),
            pl.BlockSpec(memory_space=pltpu.SMEM),
        ],
        out_specs=[
            pl.BlockSpec((1, B, H), lambda t: (t, 0, 0)),
            pl.BlockSpec((1, B, H), lambda t: (t, 0, 0)),
        ],
        out_shape=[
            jax.ShapeDtypeStruct((T, B, H), jnp.float32),
            jax.ShapeDtypeStruct((T, B, H), jnp.float32),
        ],
        scratch_shapes=[
            pltpu.VMEM((T + 1, B, 2 * H), jnp.float32),
            pltpu.VMEM((2, B, 2 * H), jnp.float32),
        ],
    )(xp, W, U, b2, li, ri)

    zeros_page = jnp.zeros((256, H), jnp.float32)
    h_flat, c_flat = _make_sc_expand(B, S, T, H)(
        h_comp.reshape(T * B, H), c_comp.reshape(T * B, H),
        src_flat, tgt_flat, zeros_page)
    return (h_flat.reshape(B, S, H), c_flat.reshape(B, S, H))
